# Initial kernel scaffold; baseline (speedup 1.0000x reference)
import jax
import jax.numpy as jnp
from jax.experimental import pallas as pl


def kernel(L_indices, L_values, H, K_channel_wise):
    raise NotImplementedError("write your pallas kernel here")


# speedup vs baseline: 4.4351x; 4.4351x over previous
import functools

import jax
import jax.numpy as jnp
from jax import lax
from jax.experimental import pallas as pl
from jax.experimental.pallas import tpu as pltpu
from jax.experimental.pallas import tpu_sc as plsc

NC = 2
NS = 16
L_LANES = 16


def _pick_chunk(ew: int) -> int:
    for b in range(128, 0, -16):
        if ew % b == 0:
            return b
    return 0


@functools.lru_cache(maxsize=None)
def _scatter_fn(n: int, d: int, e: int):
    nw = NC * NS
    assert e % nw == 0, e
    ew = e // nw
    b = _pick_chunk(ew)
    assert b > 0, ew
    n_chunks = ew // b
    rb = 8
    for cand in range(8, 4096, 8):
        if n % cand == 0 and cand * d * 4 <= 100 * 1024:
            rb = cand
    n_rblocks = n // rb
    rblocks_per_tile = -(-n_rblocks // NS)
    assert d % L_LANES == 0, d
    dv = d // L_LANES

    mesh = plsc.VectorSubcoreMesh(core_axis_name="c", subcore_axis_name="s")

    def body(row_hbm, col_hbm, val_hbm, h_hbm, k_hbm,
             p0_hbm, p1_hbm,
             acc, colbuf, rowbuf, valbuf, gbuf, dbuf, kbuf, gsem):
        cid = lax.axis_index("c")
        sid = lax.axis_index("s")
        wid = cid * NS + sid

        pltpu.sync_copy(k_hbm, kbuf)
        kv = [kbuf[pl.ds(j * L_LANES, L_LANES)] for j in range(dv)]

        zero = jnp.zeros((L_LANES,), jnp.float32)

        def zrow(r, _):
            for j in range(dv):
                dbuf[r, pl.ds(j * L_LANES, L_LANES)] = zero
            return 0

        lax.fori_loop(0, rb, zrow, 0, unroll=False)

        def zblk(t, _):
            blk = sid + NS * t

            @pl.when(blk < n_rblocks)
            def _():
                pltpu.sync_copy(dbuf, acc.at[pl.ds(blk * rb, rb)])

            return 0

        lax.fori_loop(0, rblocks_per_tile, zblk, 0, unroll=False)
        plsc.subcore_barrier()

        ebase = wid * ew

        def chunk(i, _):
            base = ebase + i * b
            pltpu.sync_copy(col_hbm.at[pl.ds(base, b)], colbuf)
            pltpu.sync_copy(row_hbm.at[pl.ds(base, b)], rowbuf)
            pltpu.sync_copy(val_hbm.at[pl.ds(base, b)], valbuf)
            pltpu.async_copy(h_hbm.at[colbuf], gbuf, gsem).wait()

            def scale(g, _):
                vv = valbuf[pl.ds(g * L_LANES, L_LANES)]
                for lane in range(L_LANES):
                    s = vv[lane]
                    ei = g * L_LANES + lane
                    for j in range(dv):
                        sl = pl.ds(j * L_LANES, L_LANES)
                        gbuf[ei, sl] = gbuf[ei, sl] * kv[j] * s
                return 0

            lax.fori_loop(0, b // L_LANES, scale, 0, unroll=False)
            pltpu.sync_copy(gbuf, acc.at[rowbuf], add=True)
            return 0

        lax.fori_loop(0, n_chunks, chunk, 0, unroll=False)
        plsc.subcore_barrier()

        def drain(t, _):
            blk = sid + NS * t

            @pl.when(blk < n_rblocks)
            def _():
                rbase = blk * rb
                pltpu.sync_copy(acc.at[pl.ds(rbase, rb)], dbuf)

                @pl.when(cid == 0)
                def _():
                    pltpu.sync_copy(dbuf, p0_hbm.at[pl.ds(rbase, rb)])

                @pl.when(cid == 1)
                def _():
                    pltpu.sync_copy(dbuf, p1_hbm.at[pl.ds(rbase, rb)])

            return 0

        lax.fori_loop(0, rblocks_per_tile, drain, 0, unroll=False)

    return pl.kernel(
        body,
        out_type=(jax.ShapeDtypeStruct((n, d), jnp.float32),
                  jax.ShapeDtypeStruct((n, d), jnp.float32)),
        mesh=mesh,
        scratch_types=[
            pltpu.VMEM_SHARED((n, d), jnp.float32),
            pltpu.VMEM((b,), jnp.int32),
            pltpu.VMEM((b,), jnp.int32),
            pltpu.VMEM((b,), jnp.float32),
            pltpu.VMEM((b, d), jnp.float32),
            pltpu.VMEM((rb, d), jnp.float32),
            pltpu.VMEM((d,), jnp.float32),
            pltpu.SemaphoreType.DMA,
        ],
    )


@functools.lru_cache(maxsize=None)
def _combine_fn(n: int, d: int):
    blk = n
    for cand in range(1000, 7, -8):
        if n % cand == 0:
            blk = cand
            break
    grid = n // blk

    def body(h_ref, p0_ref, p1_ref, o_ref):
        o_ref[...] = h_ref[...] - p0_ref[...] - p1_ref[...]

    spec = pl.BlockSpec((blk, d), lambda i: (i, 0))
    return pl.pallas_call(
        body,
        out_shape=jax.ShapeDtypeStruct((n, d), jnp.float32),
        grid=(grid,),
        in_specs=[spec, spec, spec],
        out_specs=spec,
    )


def kernel(L_indices, L_values, H, K_channel_wise):
    n, d = H.shape
    e = L_values.shape[0]
    row = L_indices[0]
    col = L_indices[1]
    k_flat = K_channel_wise.reshape(d)
    p0, p1 = _scatter_fn(n, d, e)(row, col, L_values, H, k_flat)
    return _combine_fn(n, d)(H, p0, p1)


# speedup vs baseline: 12.1660x; 2.7431x over previous
import functools

import jax
import jax.numpy as jnp
from jax import lax
from jax.experimental import pallas as pl
from jax.experimental.pallas import tpu as pltpu
from jax.experimental.pallas import tpu_sc as plsc

NC = 2
NS = 16
L_LANES = 16
NBUF = 4


def _pick_chunk(ew: int) -> int:
    for b in range(128, 0, -16):
        if ew % b == 0:
            return b
    return 0


@functools.lru_cache(maxsize=None)
def _scatter_fn(n: int, d: int, e: int):
    nw = NC * NS
    assert e % nw == 0, e
    ew = e // nw
    b = _pick_chunk(ew)
    assert b > 0, ew
    s_chunks = ew // b
    rb = 8
    for cand in range(8, 4096, 8):
        if n % cand == 0 and cand * d * 4 <= 20 * 1024:
            rb = cand
    n_rblocks = n // rb
    rblocks_per_tile = -(-n_rblocks // NS)
    assert d % L_LANES == 0, d
    dv = d // L_LANES
    n_groups = -(-s_chunks // NBUF)

    mesh = plsc.VectorSubcoreMesh(core_axis_name="c", subcore_axis_name="s")

    def body(row_hbm, col_hbm, val_hbm, h_hbm, k_hbm,
             p0_hbm, p1_hbm,
             acc, colbufs, rowbufs, valbufs, dbuf, kbuf,
             gbufs, isems, gsems, ssems):
        cid = lax.axis_index("c")
        sid = lax.axis_index("s")
        wid = cid * NS + sid
        ebase = wid * ew

        pltpu.sync_copy(k_hbm, kbuf)
        kv = [kbuf[pl.ds(j * L_LANES, L_LANES)] for j in range(dv)]

        zero = jnp.zeros((L_LANES,), jnp.float32)

        def zrow(r, _):
            for j in range(dv):
                dbuf[r, pl.ds(j * L_LANES, L_LANES)] = zero
            return 0

        lax.fori_loop(0, rb, zrow, 0, unroll=False)

        def zblk(t, _):
            blk = sid + NS * t

            @pl.when(blk < n_rblocks)
            def _():
                pltpu.sync_copy(dbuf, acc.at[pl.ds(blk * rb, rb)])

            return 0

        lax.fori_loop(0, rblocks_per_tile, zblk, 0, unroll=False)
        plsc.subcore_barrier()

        def issue_idx(c, w):
            base = ebase + c * b
            pltpu.async_copy(col_hbm.at[pl.ds(base, b)], colbufs.at[w],
                             isems.at[w])
            pltpu.async_copy(row_hbm.at[pl.ds(base, b)], rowbufs.at[w],
                             isems.at[w])
            pltpu.async_copy(val_hbm.at[pl.ds(base, b)], valbufs.at[w],
                             isems.at[w])

        def wait_idx(c, w):
            base = ebase + c * b
            pltpu.make_async_copy(col_hbm.at[pl.ds(base, b)], colbufs.at[w],
                                  isems.at[w]).wait()
            pltpu.make_async_copy(row_hbm.at[pl.ds(base, b)], rowbufs.at[w],
                                  isems.at[w]).wait()
            pltpu.make_async_copy(val_hbm.at[pl.ds(base, b)], valbufs.at[w],
                                  isems.at[w]).wait()

        def issue_gather(w):
            pltpu.async_copy(h_hbm.at[colbufs.at[w]], gbufs.at[w],
                             gsems.at[w])

        def wait_gather(w):
            pltpu.make_async_copy(h_hbm.at[colbufs.at[w]], gbufs.at[w],
                                  gsems.at[w]).wait()

        def issue_scatter(w):
            pltpu.async_copy(gbufs.at[w], acc.at[rowbufs.at[w]],
                             ssems.at[w], add=True)

        def wait_scatter(w):
            pltpu.make_async_copy(gbufs.at[w], acc.at[rowbufs.at[w]],
                                  ssems.at[w]).wait()

        issue_idx(0, 0)
        if s_chunks > 1:
            issue_idx(1, 1)
        wait_idx(0, 0)
        issue_gather(0)

        def group(g, _):
            for u in range(NBUF):
                c = g * NBUF + u

                @pl.when(c < s_chunks)
                def _(c=c, u=u):
                    u1 = (u + 1) % NBUF
                    u2 = (u + 2) % NBUF

                    @pl.when(c >= 2)
                    def _():
                        wait_scatter(u2)

                    @pl.when(c + 2 < s_chunks)
                    def _():
                        issue_idx(c + 2, u2)

                    @pl.when(c + 1 < s_chunks)
                    def _():
                        wait_idx(c + 1, u1)
                        issue_gather(u1)

                    wait_gather(u)

                    def scale(grp, _):
                        vv = valbufs[u, pl.ds(grp * L_LANES, L_LANES)]
                        for lane in range(L_LANES):
                            s = vv[lane]
                            ei = grp * L_LANES + lane
                            for j in range(dv):
                                sl = pl.ds(j * L_LANES, L_LANES)
                                gbufs[u, ei, sl] = gbufs[u, ei, sl] * kv[j] * s
                        return 0

                    lax.fori_loop(0, b // L_LANES, scale, 0, unroll=False)
                    issue_scatter(u)

            return 0

        lax.fori_loop(0, n_groups, group, 0, unroll=False)
        if s_chunks > 1:
            wait_scatter((s_chunks - 2) % NBUF)
        wait_scatter((s_chunks - 1) % NBUF)
        plsc.subcore_barrier()

        def drain(t, _):
            blk = sid + NS * t

            @pl.when(blk < n_rblocks)
            def _():
                rbase = blk * rb
                pltpu.sync_copy(acc.at[pl.ds(rbase, rb)], dbuf)

                @pl.when(cid == 0)
                def _():
                    pltpu.sync_copy(dbuf, p0_hbm.at[pl.ds(rbase, rb)])

                @pl.when(cid == 1)
                def _():
                    pltpu.sync_copy(dbuf, p1_hbm.at[pl.ds(rbase, rb)])

            return 0

        lax.fori_loop(0, rblocks_per_tile, drain, 0, unroll=False)

    return pl.kernel(
        body,
        out_type=(jax.ShapeDtypeStruct((n, d), jnp.float32),
                  jax.ShapeDtypeStruct((n, d), jnp.float32)),
        mesh=mesh,
        scratch_types=[
            pltpu.VMEM_SHARED((n, d), jnp.float32),
            pltpu.VMEM((NBUF, b), jnp.int32),
            pltpu.VMEM((NBUF, b), jnp.int32),
            pltpu.VMEM((NBUF, b), jnp.float32),
            pltpu.VMEM((rb, d), jnp.float32),
            pltpu.VMEM((d,), jnp.float32),
            pltpu.VMEM((NBUF, b, d), jnp.float32),
            pltpu.SemaphoreType.DMA((NBUF,)),
            pltpu.SemaphoreType.DMA((NBUF,)),
            pltpu.SemaphoreType.DMA((NBUF,)),
        ],
    )


@functools.lru_cache(maxsize=None)
def _combine_fn(n: int, d: int):
    blk = n
    for cand in range(1000, 7, -8):
        if n % cand == 0:
            blk = cand
            break
    grid = n // blk

    def body(h_ref, p0_ref, p1_ref, o_ref):
        o_ref[...] = h_ref[...] - p0_ref[...] - p1_ref[...]

    spec = pl.BlockSpec((blk, d), lambda i: (i, 0))
    return pl.pallas_call(
        body,
        out_shape=jax.ShapeDtypeStruct((n, d), jnp.float32),
        grid=(grid,),
        in_specs=[spec, spec, spec],
        out_specs=spec,
    )


def kernel(L_indices, L_values, H, K_channel_wise):
    n, d = H.shape
    e = L_values.shape[0]
    row = L_indices[0]
    col = L_indices[1]
    k_flat = K_channel_wise.reshape(d)
    p0, p1 = _scatter_fn(n, d, e)(row, col, L_values, H, k_flat)
    return _combine_fn(n, d)(H, p0, p1)


# speedup vs baseline: 12.2443x; 1.0064x over previous
import functools

import jax
import jax.numpy as jnp
from jax import lax
from jax.experimental import pallas as pl
from jax.experimental.pallas import tpu as pltpu
from jax.experimental.pallas import tpu_sc as plsc

NC = 2
NS = 16
L_LANES = 16
NBUF = 4


def _pick_chunk(ew: int) -> int:
    for b in range(128, 0, -16):
        if ew % b == 0:
            return b
    return 0


@functools.lru_cache(maxsize=None)
def _scatter_fn(n: int, d: int, e: int):
    nw = NC * NS
    assert e % nw == 0, e
    ew = e // nw
    b = _pick_chunk(ew)
    assert b > 0, ew
    s_chunks = ew // b
    rb = 8
    for cand in range(8, 4096, 8):
        if n % cand == 0 and cand * d * 4 <= 128 * 1024:
            rb = cand
    n_rblocks = n // rb
    rblocks_per_tile = -(-n_rblocks // NS)
    assert d % L_LANES == 0, d
    dv = d // L_LANES
    n_groups = -(-s_chunks // NBUF)

    mesh = plsc.VectorSubcoreMesh(core_axis_name="c", subcore_axis_name="s")

    def body(row_hbm, col_hbm, val_hbm, h_hbm, z_hbm,
             p0_hbm, p1_hbm,
             acc, colbufs, rowbufs, valbufs,
             gbufs, isems, gsems, ssems, psem):
        cid = lax.axis_index("c")
        sid = lax.axis_index("s")
        wid = cid * NS + sid
        ebase = wid * ew

        def issue_idx(c, w):
            base = ebase + c * b
            pltpu.async_copy(col_hbm.at[pl.ds(base, b)], colbufs.at[w],
                             isems.at[w])
            pltpu.async_copy(row_hbm.at[pl.ds(base, b)], rowbufs.at[w],
                             isems.at[w])
            pltpu.async_copy(val_hbm.at[pl.ds(base, b)], valbufs.at[w],
                             isems.at[w])

        def wait_idx(c, w):
            base = ebase + c * b
            pltpu.make_async_copy(col_hbm.at[pl.ds(base, b)], colbufs.at[w],
                                  isems.at[w]).wait()
            pltpu.make_async_copy(row_hbm.at[pl.ds(base, b)], rowbufs.at[w],
                                  isems.at[w]).wait()
            pltpu.make_async_copy(val_hbm.at[pl.ds(base, b)], valbufs.at[w],
                                  isems.at[w]).wait()

        def issue_gather(w):
            pltpu.async_copy(h_hbm.at[colbufs.at[w]], gbufs.at[w],
                             gsems.at[w])

        def wait_gather(w):
            pltpu.make_async_copy(h_hbm.at[colbufs.at[w]], gbufs.at[w],
                                  gsems.at[w]).wait()

        def issue_scatter(w):
            pltpu.async_copy(gbufs.at[w], acc.at[rowbufs.at[w]],
                             ssems.at[w], add=True)

        def wait_scatter(w):
            pltpu.make_async_copy(gbufs.at[w], acc.at[rowbufs.at[w]],
                                  ssems.at[w]).wait()

        issue_idx(0, 0)
        if s_chunks > 1:
            issue_idx(1, 1)

        for t in range(rblocks_per_tile):
            blk = sid + NS * t

            @pl.when(blk < n_rblocks)
            def _(blk=blk):
                pltpu.async_copy(z_hbm.at[pl.ds(blk * rb, rb)],
                                 acc.at[pl.ds(blk * rb, rb)], psem)

        for t in range(rblocks_per_tile):
            blk = sid + NS * t

            @pl.when(blk < n_rblocks)
            def _(blk=blk):
                pltpu.make_async_copy(z_hbm.at[pl.ds(blk * rb, rb)],
                                      acc.at[pl.ds(blk * rb, rb)],
                                      psem).wait()

        plsc.subcore_barrier()

        wait_idx(0, 0)
        issue_gather(0)

        def group(g, _):
            for u in range(NBUF):
                c = g * NBUF + u

                @pl.when(c < s_chunks)
                def _(c=c, u=u):
                    u1 = (u + 1) % NBUF
                    u2 = (u + 2) % NBUF

                    @pl.when(c >= 2)
                    def _():
                        wait_scatter(u2)

                    @pl.when(c + 2 < s_chunks)
                    def _():
                        issue_idx(c + 2, u2)

                    @pl.when(c + 1 < s_chunks)
                    def _():
                        wait_idx(c + 1, u1)
                        issue_gather(u1)

                    wait_gather(u)

                    def scale(grp, _):
                        vv = valbufs[u, pl.ds(grp * L_LANES, L_LANES)]
                        for lane in range(L_LANES):
                            s = vv[lane]
                            ei = grp * L_LANES + lane
                            for j in range(dv):
                                sl = pl.ds(j * L_LANES, L_LANES)
                                gbufs[u, ei, sl] = gbufs[u, ei, sl] * s
                        return 0

                    lax.fori_loop(0, b // L_LANES, scale, 0, unroll=False)
                    issue_scatter(u)

            return 0

        lax.fori_loop(0, n_groups, group, 0, unroll=False)
        if s_chunks > 1:
            wait_scatter((s_chunks - 2) % NBUF)
        wait_scatter((s_chunks - 1) % NBUF)
        plsc.subcore_barrier()

        for t in range(rblocks_per_tile):
            blk = sid + NS * t

            @pl.when(blk < n_rblocks)
            def _(blk=blk):
                sl = pl.ds(blk * rb, rb)

                @pl.when(cid == 0)
                def _():
                    pltpu.async_copy(acc.at[sl], p0_hbm.at[sl], psem)

                @pl.when(cid == 1)
                def _():
                    pltpu.async_copy(acc.at[sl], p1_hbm.at[sl], psem)

        for t in range(rblocks_per_tile):
            blk = sid + NS * t

            @pl.when(blk < n_rblocks)
            def _(blk=blk):
                sl = pl.ds(blk * rb, rb)

                @pl.when(cid == 0)
                def _():
                    pltpu.make_async_copy(acc.at[sl], p0_hbm.at[sl],
                                          psem).wait()

                @pl.when(cid == 1)
                def _():
                    pltpu.make_async_copy(acc.at[sl], p1_hbm.at[sl],
                                          psem).wait()

    return pl.kernel(
        body,
        out_type=(jax.ShapeDtypeStruct((n, d), jnp.float32),
                  jax.ShapeDtypeStruct((n, d), jnp.float32)),
        mesh=mesh,
        scratch_types=[
            pltpu.VMEM_SHARED((n, d), jnp.float32),
            pltpu.VMEM((NBUF, b), jnp.int32),
            pltpu.VMEM((NBUF, b), jnp.int32),
            pltpu.VMEM((NBUF, b), jnp.float32),
            pltpu.VMEM((NBUF, b, d), jnp.float32),
            pltpu.SemaphoreType.DMA((NBUF,)),
            pltpu.SemaphoreType.DMA((NBUF,)),
            pltpu.SemaphoreType.DMA((NBUF,)),
            pltpu.SemaphoreType.DMA,
        ],
    )


@functools.lru_cache(maxsize=None)
def _combine_fn(n: int, d: int):
    blk = n
    for cand in range(1000, 7, -8):
        if n % cand == 0:
            blk = cand
            break
    grid = n // blk

    def body(h_ref, p0_ref, p1_ref, k_ref, o_ref):
        o_ref[...] = h_ref[...] - (p0_ref[...] + p1_ref[...]) * k_ref[...]

    spec = pl.BlockSpec((blk, d), lambda i: (i, 0))
    kspec = pl.BlockSpec((1, d), lambda i: (0, 0))
    return pl.pallas_call(
        body,
        out_shape=jax.ShapeDtypeStruct((n, d), jnp.float32),
        grid=(grid,),
        in_specs=[spec, spec, spec, kspec],
        out_specs=spec,
    )


def kernel(L_indices, L_values, H, K_channel_wise):
    n, d = H.shape
    e = L_values.shape[0]
    row = L_indices[0]
    col = L_indices[1]
    zeros = jnp.zeros((n, d), jnp.float32)
    p0, p1 = _scatter_fn(n, d, e)(row, col, L_values, H, zeros)
    return _combine_fn(n, d)(H, p0, p1, K_channel_wise)


# speedup vs baseline: 12.3787x; 1.0110x over previous
import functools

import jax
import jax.numpy as jnp
from jax import lax
from jax.experimental import pallas as pl
from jax.experimental.pallas import tpu as pltpu
from jax.experimental.pallas import tpu_sc as plsc

NC = 2
NS = 16
L_LANES = 16
NBUF = 4
NI = 8
GROUP = 8


def _pick_chunk(ew: int) -> int:
    for b in range(128, 0, -16):
        if ew % b == 0:
            return b
    return 0


@functools.lru_cache(maxsize=None)
def _scatter_fn(n: int, d: int, e: int):
    nw = NC * NS
    assert e % nw == 0, e
    ew = e // nw
    b = _pick_chunk(ew)
    assert b > 0, ew
    s_chunks = ew // b
    rb = 8
    for cand in range(8, 4096, 8):
        if n % cand == 0 and cand * d * 4 <= 128 * 1024:
            rb = cand
    n_rblocks = n // rb
    rblocks_per_tile = -(-n_rblocks // NS)
    assert d % L_LANES == 0, d
    dv = d // L_LANES
    n_groups = -(-s_chunks // GROUP)

    mesh = plsc.VectorSubcoreMesh(core_axis_name="c", subcore_axis_name="s")

    def body(row_hbm, col_hbm, val_hbm, h_hbm, z_hbm,
             p0_hbm, p1_hbm,
             acc, colbufs, rowbufs, valbufs,
             gbufs, isems, gsems, ssems, psem):
        cid = lax.axis_index("c")
        sid = lax.axis_index("s")
        wid = cid * NS + sid
        ebase = wid * ew

        def issue_idx(c, w):
            base = ebase + c * b
            pltpu.async_copy(col_hbm.at[pl.ds(base, b)], colbufs.at[w],
                             isems.at[w])
            pltpu.async_copy(row_hbm.at[pl.ds(base, b)], rowbufs.at[w],
                             isems.at[w])
            pltpu.async_copy(val_hbm.at[pl.ds(base, b)], valbufs.at[w],
                             isems.at[w])

        def wait_idx(c, w):
            base = ebase + c * b
            pltpu.make_async_copy(col_hbm.at[pl.ds(base, b)], colbufs.at[w],
                                  isems.at[w]).wait()
            pltpu.make_async_copy(row_hbm.at[pl.ds(base, b)], rowbufs.at[w],
                                  isems.at[w]).wait()
            pltpu.make_async_copy(val_hbm.at[pl.ds(base, b)], valbufs.at[w],
                                  isems.at[w]).wait()

        def issue_gather(wg, wi):
            pltpu.async_copy(h_hbm.at[colbufs.at[wi]], gbufs.at[wg],
                             gsems.at[wg])

        def wait_gather(wg, wi):
            pltpu.make_async_copy(h_hbm.at[colbufs.at[wi]], gbufs.at[wg],
                                  gsems.at[wg]).wait()

        def issue_scatter(wg, wi):
            pltpu.async_copy(gbufs.at[wg], acc.at[rowbufs.at[wi]],
                             ssems.at[wg], add=True)

        def wait_scatter(wg, wi):
            pltpu.make_async_copy(gbufs.at[wg], acc.at[rowbufs.at[wi]],
                                  ssems.at[wg]).wait()

        for p in range(min(4, s_chunks)):
            issue_idx(p, p)

        for t in range(rblocks_per_tile):
            blk = sid + NS * t

            @pl.when(blk < n_rblocks)
            def _(blk=blk):
                pltpu.async_copy(z_hbm.at[pl.ds(blk * rb, rb)],
                                 acc.at[pl.ds(blk * rb, rb)], psem)

        for t in range(rblocks_per_tile):
            blk = sid + NS * t

            @pl.when(blk < n_rblocks)
            def _(blk=blk):
                pltpu.make_async_copy(z_hbm.at[pl.ds(blk * rb, rb)],
                                      acc.at[pl.ds(blk * rb, rb)],
                                      psem).wait()

        plsc.subcore_barrier()

        wait_idx(0, 0)
        issue_gather(0, 0)
        if s_chunks > 1:
            wait_idx(1, 1)
            issue_gather(1, 1)

        def group(g, _):
            for k in range(GROUP):
                c = g * GROUP + k

                @pl.when(c < s_chunks)
                def _(c=c, k=k):
                    kg = k % NBUF

                    @pl.when(c >= 2)
                    def _():
                        wait_scatter((k - 2) % NBUF, (k - 2) % NI)

                    @pl.when(c + 4 < s_chunks)
                    def _():
                        issue_idx(c + 4, (k + 4) % NI)

                    @pl.when(c + 2 < s_chunks)
                    def _():
                        wait_idx(c + 2, (k + 2) % NI)
                        issue_gather((k + 2) % NBUF, (k + 2) % NI)

                    wait_gather(kg, k % NI)

                    def scale(grp, _):
                        vv = valbufs[k % NI, pl.ds(grp * L_LANES, L_LANES)]
                        for lane in range(L_LANES):
                            s = vv[lane]
                            ei = grp * L_LANES + lane
                            for j in range(dv):
                                sl = pl.ds(j * L_LANES, L_LANES)
                                gbufs[kg, ei, sl] = gbufs[kg, ei, sl] * s
                        return 0

                    lax.fori_loop(0, b // L_LANES, scale, 0, unroll=False)
                    issue_scatter(kg, k % NI)

            return 0

        lax.fori_loop(0, n_groups, group, 0, unroll=False)
        if s_chunks > 1:
            wait_scatter((s_chunks - 2) % NBUF, (s_chunks - 2) % NI)
        wait_scatter((s_chunks - 1) % NBUF, (s_chunks - 1) % NI)
        plsc.subcore_barrier()

        for t in range(rblocks_per_tile):
            blk = sid + NS * t

            @pl.when(blk < n_rblocks)
            def _(blk=blk):
                sl = pl.ds(blk * rb, rb)

                @pl.when(cid == 0)
                def _():
                    pltpu.async_copy(acc.at[sl], p0_hbm.at[sl], psem)

                @pl.when(cid == 1)
                def _():
                    pltpu.async_copy(acc.at[sl], p1_hbm.at[sl], psem)

        for t in range(rblocks_per_tile):
            blk = sid + NS * t

            @pl.when(blk < n_rblocks)
            def _(blk=blk):
                sl = pl.ds(blk * rb, rb)

                @pl.when(cid == 0)
                def _():
                    pltpu.make_async_copy(acc.at[sl], p0_hbm.at[sl],
                                          psem).wait()

                @pl.when(cid == 1)
                def _():
                    pltpu.make_async_copy(acc.at[sl], p1_hbm.at[sl],
                                          psem).wait()

    return pl.kernel(
        body,
        out_type=(jax.ShapeDtypeStruct((n, d), jnp.float32),
                  jax.ShapeDtypeStruct((n, d), jnp.float32)),
        mesh=mesh,
        scratch_types=[
            pltpu.VMEM_SHARED((n, d), jnp.float32),
            pltpu.VMEM((NI, b), jnp.int32),
            pltpu.VMEM((NI, b), jnp.int32),
            pltpu.VMEM((NI, b), jnp.float32),
            pltpu.VMEM((NBUF, b, d), jnp.float32),
            pltpu.SemaphoreType.DMA((NI,)),
            pltpu.SemaphoreType.DMA((NBUF,)),
            pltpu.SemaphoreType.DMA((NBUF,)),
            pltpu.SemaphoreType.DMA,
        ],
    )


@functools.lru_cache(maxsize=None)
def _combine_fn(n: int, d: int):
    blk = n
    for cand in range(1000, 7, -8):
        if n % cand == 0:
            blk = cand
            break
    grid = n // blk

    def body(h_ref, p0_ref, p1_ref, k_ref, o_ref):
        o_ref[...] = h_ref[...] - (p0_ref[...] + p1_ref[...]) * k_ref[...]

    spec = pl.BlockSpec((blk, d), lambda i: (i, 0))
    kspec = pl.BlockSpec((1, d), lambda i: (0, 0))
    return pl.pallas_call(
        body,
        out_shape=jax.ShapeDtypeStruct((n, d), jnp.float32),
        grid=(grid,),
        in_specs=[spec, spec, spec, kspec],
        out_specs=spec,
    )


def kernel(L_indices, L_values, H, K_channel_wise):
    n, d = H.shape
    e = L_values.shape[0]
    row = L_indices[0]
    col = L_indices[1]
    zeros = jnp.zeros((n, d), jnp.float32)
    p0, p1 = _scatter_fn(n, d, e)(row, col, L_values, H, zeros)
    return _combine_fn(n, d)(H, p0, p1, K_channel_wise)
